# baseline (device time: 468893 ns/iter reference)
import jax
import jax.numpy as jnp
from jax import lax
from jax.experimental import pallas as pl
from jax.experimental.pallas import tpu as pltpu

N_DEV = 4
M_SH = 2048
K = 8192
N = 4096
N_PER = N // N_DEV
MH = M_SH // 2
WT = 256
NWT = K // WT
DT = 1024
SUB = DT // WT
XT = 128
NXT = K // XT


def kernel(x, w_mat):
    def body(x_hbm, w_hbm, out_hbm, recv_hbm,
             x_bf, w_stage, w_bf, acc, x_stage, send_buf,
             copy_sem, wsems, xsems, send_sems, recv_sems):
        my = lax.axis_index("i")

        def local_copy(src, dst):
            cp = pltpu.make_async_copy(src, dst, copy_sem)
            cp.start()
            cp.wait()

        pending = {0: None, 1: None}

        for h in range(2):
            def xcp(kt, slot):
                return pltpu.make_async_copy(
                    x_hbm.at[pl.ds(h * MH, MH), pl.ds(kt * XT, XT)],
                    x_stage.at[slot], xsems.at[slot])

            xcp(0, 0).start()

            def xconv(kt, carry):
                slot = lax.rem(kt, 2)

                @pl.when(kt + 1 < NXT)
                def _():
                    xcp(kt + 1, 1 - slot).start()

                xcp(kt, slot).wait()
                x_bf[:, pl.ds(kt * XT, XT)] = (
                    x_stage[slot].astype(jnp.bfloat16))
                return carry

            lax.fori_loop(0, NXT, xconv, 0)

            for t in (1, 2, 3, 0):
                j = (my + t) % N_DEV
                col = pl.ds(j * N_PER, N_PER)
                acc[...] = jnp.zeros((MH, N_PER), jnp.float32)

                def wcp(kt, slot, col=col):
                    return pltpu.make_async_copy(
                        w_hbm.at[pl.ds(kt * WT, WT), col],
                        w_stage.at[slot], wsems.at[slot])

                wcp(0, 0).start()
                wcp(1, 1).start()

                def kstep(kk, carry, wcp=wcp):
                    for q in range(SUB):
                        kt = SUB * kk + q
                        slot = lax.rem(kt, 2)
                        wcp(kt, slot).wait()
                        w_bf[pl.ds(q * WT, WT), :] = (
                            w_stage[slot].astype(jnp.bfloat16))

                        @pl.when(kt + 2 < NWT)
                        def _():
                            wcp(kt + 2, slot).start()
                    acc[...] = acc[...] + jnp.dot(
                        x_bf[:, pl.ds(kk * DT, DT)], w_bf[...],
                        preferred_element_type=jnp.float32)
                    return carry

                lax.fori_loop(0, K // DT, kstep, 0)

                if t == 0:
                    acc[...] = jnp.maximum(acc[...], 0.0)
                    local_copy(acc,
                               out_hbm.at[pl.ds(my * M_SH + h * MH, MH), :])
                else:
                    s = (t - 1) % 2
                    if pending[s] is not None:
                        pending[s].wait_send()
                    send_buf[s] = jnp.maximum(acc[...], 0.0).astype(
                        jnp.bfloat16)
                    rdma = pltpu.make_async_remote_copy(
                        src_ref=send_buf.at[s],
                        dst_ref=recv_hbm.at[3 - t, h],
                        send_sem=send_sems.at[s],
                        recv_sem=recv_sems.at[3 - t, h],
                        device_id=(j,),
                        device_id_type=pl.DeviceIdType.MESH,
                    )
                    rdma.start()
                    pending[s] = rdma

        for s in (0, 1):
            if pending[s] is not None:
                pending[s].wait_send()
                pending[s] = None

        for h in range(2):
            for u in (3, 2, 1):
                s = (my + u) % N_DEV
                recv = pltpu.make_async_remote_copy(
                    src_ref=send_buf.at[0],
                    dst_ref=recv_hbm.at[u - 1, h],
                    send_sem=send_sems.at[0],
                    recv_sem=recv_sems.at[u - 1, h],
                    device_id=(my,),
                    device_id_type=pl.DeviceIdType.MESH,
                )
                recv.wait_recv()
                local_copy(recv_hbm.at[u - 1, h], send_buf.at[0])
                acc[...] = send_buf[0].astype(jnp.float32)
                local_copy(acc,
                           out_hbm.at[pl.ds(s * M_SH + h * MH, MH), :])

    out, _ = pl.pallas_call(
        body,
        out_shape=[
            jax.ShapeDtypeStruct((N_DEV * M_SH, N_PER), jnp.float32),
            jax.ShapeDtypeStruct((3, 2, MH, N_PER), jnp.bfloat16),
        ],
        in_specs=[
            pl.BlockSpec(memory_space=pltpu.MemorySpace.HBM),
            pl.BlockSpec(memory_space=pltpu.MemorySpace.HBM),
        ],
        out_specs=[
            pl.BlockSpec(memory_space=pltpu.MemorySpace.HBM),
            pl.BlockSpec(memory_space=pltpu.MemorySpace.HBM),
        ],
        scratch_shapes=[
            pltpu.VMEM((MH, K), jnp.bfloat16),
            pltpu.VMEM((2, WT, N_PER), jnp.float32),
            pltpu.VMEM((DT, N_PER), jnp.bfloat16),
            pltpu.VMEM((MH, N_PER), jnp.float32),
            pltpu.VMEM((2, MH, XT), jnp.float32),
            pltpu.VMEM((2, MH, N_PER), jnp.bfloat16),
            pltpu.SemaphoreType.DMA,
            pltpu.SemaphoreType.DMA((2,)),
            pltpu.SemaphoreType.DMA((2,)),
            pltpu.SemaphoreType.DMA((2,)),
            pltpu.SemaphoreType.DMA((3, 2)),
        ],
    )(x, w_mat)
    return out


# device time: 408029 ns/iter; 1.1492x vs baseline; 1.1492x over previous
import jax
import jax.numpy as jnp
from jax import lax
from jax.experimental import pallas as pl
from jax.experimental.pallas import tpu as pltpu

N_DEV = 4
M_SH = 2048
K = 8192
N = 4096
N_PER = N // N_DEV
MH = M_SH // 2
WT = 512
NWT = K // WT
DT = 1024
SUB = DT // WT
XT = 128
NXT = K // XT


def kernel(x, w_mat):
    def body(x_hbm, w_hbm, out_hbm, recv_hbm,
             x_bf, w_stage, w_bf, acc, x_stage, send_buf,
             copy_sem, wsems, xsems, send_sem, recv_sems):
        my = lax.axis_index("i")

        def local_copy(src, dst):
            cp = pltpu.make_async_copy(src, dst, copy_sem)
            cp.start()
            cp.wait()

        pending = [None]

        for h in range(2):
            def xcp(kt, slot):
                return pltpu.make_async_copy(
                    x_hbm.at[pl.ds(h * MH, MH), pl.ds(kt * XT, XT)],
                    x_stage.at[slot], xsems.at[slot])

            xcp(0, 0).start()

            def xconv(kt, carry):
                slot = lax.rem(kt, 2)

                @pl.when(kt + 1 < NXT)
                def _():
                    xcp(kt + 1, 1 - slot).start()

                xcp(kt, slot).wait()
                x_bf[:, pl.ds(kt * XT, XT)] = (
                    x_stage[slot].astype(jnp.bfloat16))
                return carry

            lax.fori_loop(0, NXT, xconv, 0)

            for t in (1, 2, 3, 0):
                j = (my + t) % N_DEV
                col = pl.ds(j * N_PER, N_PER)
                acc[...] = jnp.zeros((MH, N_PER), jnp.float32)

                def wcp(kt, slot, col=col):
                    return pltpu.make_async_copy(
                        w_hbm.at[pl.ds(kt * WT, WT), col],
                        w_stage.at[slot], wsems.at[slot])

                wcp(0, 0).start()

                def kstep(kk, carry, wcp=wcp):
                    for q in range(SUB):
                        kt = SUB * kk + q
                        slot = lax.rem(kt, 2)

                        @pl.when(kt + 1 < NWT)
                        def _():
                            wcp(kt + 1, 1 - slot).start()

                        wcp(kt, slot).wait()
                        w_bf[pl.ds(q * WT, WT), :] = (
                            w_stage[slot].astype(jnp.bfloat16))
                    acc[...] = acc[...] + jnp.dot(
                        x_bf[:, pl.ds(kk * DT, DT)], w_bf[...],
                        preferred_element_type=jnp.float32)
                    return carry

                lax.fori_loop(0, K // DT, kstep, 0)

                if t == 0:
                    acc[...] = jnp.maximum(acc[...], 0.0)
                    local_copy(acc,
                               out_hbm.at[pl.ds(my * M_SH + h * MH, MH), :])
                else:
                    if pending[0] is not None:
                        pending[0].wait_send()
                    send_buf[...] = jnp.maximum(acc[...], 0.0).astype(
                        jnp.bfloat16)
                    rdma = pltpu.make_async_remote_copy(
                        src_ref=send_buf,
                        dst_ref=recv_hbm.at[3 - t, h],
                        send_sem=send_sem,
                        recv_sem=recv_sems.at[3 - t, h],
                        device_id=(j,),
                        device_id_type=pl.DeviceIdType.MESH,
                    )
                    rdma.start()
                    pending[0] = rdma

        if pending[0] is not None:
            pending[0].wait_send()
            pending[0] = None

        for h in range(2):
            for u in (3, 2, 1):
                s = (my + u) % N_DEV
                recv = pltpu.make_async_remote_copy(
                    src_ref=send_buf,
                    dst_ref=recv_hbm.at[u - 1, h],
                    send_sem=send_sem,
                    recv_sem=recv_sems.at[u - 1, h],
                    device_id=(my,),
                    device_id_type=pl.DeviceIdType.MESH,
                )
                recv.wait_recv()
                local_copy(recv_hbm.at[u - 1, h], send_buf)
                acc[...] = send_buf[...].astype(jnp.float32)
                local_copy(acc,
                           out_hbm.at[pl.ds(s * M_SH + h * MH, MH), :])

    out, _ = pl.pallas_call(
        body,
        out_shape=[
            jax.ShapeDtypeStruct((N_DEV * M_SH, N_PER), jnp.float32),
            jax.ShapeDtypeStruct((3, 2, MH, N_PER), jnp.bfloat16),
        ],
        in_specs=[
            pl.BlockSpec(memory_space=pltpu.MemorySpace.HBM),
            pl.BlockSpec(memory_space=pltpu.MemorySpace.HBM),
        ],
        out_specs=[
            pl.BlockSpec(memory_space=pltpu.MemorySpace.HBM),
            pl.BlockSpec(memory_space=pltpu.MemorySpace.HBM),
        ],
        scratch_shapes=[
            pltpu.VMEM((MH, K), jnp.bfloat16),
            pltpu.VMEM((2, WT, N_PER), jnp.float32),
            pltpu.VMEM((DT, N_PER), jnp.bfloat16),
            pltpu.VMEM((MH, N_PER), jnp.float32),
            pltpu.VMEM((2, MH, XT), jnp.float32),
            pltpu.VMEM((MH, N_PER), jnp.bfloat16),
            pltpu.SemaphoreType.DMA,
            pltpu.SemaphoreType.DMA((2,)),
            pltpu.SemaphoreType.DMA((2,)),
            pltpu.SemaphoreType.DMA,
            pltpu.SemaphoreType.DMA((3, 2)),
        ],
    )(x, w_mat)
    return out


# device time: 343275 ns/iter; 1.3659x vs baseline; 1.1886x over previous
import jax
import jax.numpy as jnp
from jax import lax
from jax.experimental import pallas as pl
from jax.experimental.pallas import tpu as pltpu

N_DEV = 4
M_SH = 2048
K = 8192
N = 4096
N_PER = N // N_DEV
MH = M_SH // 2
WT = 512
NWT = K // WT
DT = 512
SUB = DT // WT
XT = 128
NXT = K // XT


def kernel(x, w_mat):
    def body(x_hbm, w_hbm, out_hbm, recv_hbm,
             x_bf, w_stage, w_bf, acc, x_stage, send_buf,
             copy_sem, wsems, xsems, send_sems, recv_sems):
        my = lax.axis_index("i")

        def local_copy(src, dst):
            cp = pltpu.make_async_copy(src, dst, copy_sem)
            cp.start()
            cp.wait()

        pending = {0: None, 1: None}

        for h in range(2):
            def xcp(kt, slot):
                return pltpu.make_async_copy(
                    x_hbm.at[pl.ds(h * MH, MH), pl.ds(kt * XT, XT)],
                    x_stage.at[slot], xsems.at[slot])

            xcp(0, 0).start()

            def xconv(kt, carry):
                slot = lax.rem(kt, 2)

                @pl.when(kt + 1 < NXT)
                def _():
                    xcp(kt + 1, 1 - slot).start()

                xcp(kt, slot).wait()
                x_bf[:, pl.ds(kt * XT, XT)] = (
                    x_stage[slot].astype(jnp.bfloat16))
                return carry

            lax.fori_loop(0, NXT, xconv, 0)

            for t in (1, 2, 3, 0):
                j = (my + t) % N_DEV
                col = pl.ds(j * N_PER, N_PER)
                acc[...] = jnp.zeros((MH, N_PER), jnp.float32)

                def wcp(kt, slot, col=col):
                    return pltpu.make_async_copy(
                        w_hbm.at[pl.ds(kt * WT, WT), col],
                        w_stage.at[slot], wsems.at[slot])

                wcp(0, 0).start()

                def kstep(kk, carry, wcp=wcp):
                    for q in range(SUB):
                        kt = SUB * kk + q
                        slot = lax.rem(kt, 2)

                        @pl.when(kt + 1 < NWT)
                        def _():
                            wcp(kt + 1, 1 - slot).start()

                        wcp(kt, slot).wait()
                        w_bf[pl.ds(q * WT, WT), :] = (
                            w_stage[slot].astype(jnp.bfloat16))
                    acc[...] = acc[...] + jnp.dot(
                        x_bf[:, pl.ds(kk * DT, DT)], w_bf[...],
                        preferred_element_type=jnp.float32)
                    return carry

                lax.fori_loop(0, K // DT, kstep, 0)

                if t == 0:
                    acc[...] = jnp.maximum(acc[...], 0.0)
                    local_copy(acc,
                               out_hbm.at[pl.ds(my * M_SH + h * MH, MH), :])
                else:
                    s = (t - 1) % 2
                    if pending[s] is not None:
                        pending[s].wait_send()
                    send_buf[s] = jnp.maximum(acc[...], 0.0).astype(
                        jnp.bfloat16)
                    rdma = pltpu.make_async_remote_copy(
                        src_ref=send_buf.at[s],
                        dst_ref=recv_hbm.at[3 - t, h],
                        send_sem=send_sems.at[s],
                        recv_sem=recv_sems.at[3 - t, h],
                        device_id=(j,),
                        device_id_type=pl.DeviceIdType.MESH,
                    )
                    rdma.start()
                    pending[s] = rdma

        for s in (0, 1):
            if pending[s] is not None:
                pending[s].wait_send()
                pending[s] = None

        for h in range(2):
            for u in (3, 2, 1):
                s = (my + u) % N_DEV
                recv = pltpu.make_async_remote_copy(
                    src_ref=send_buf.at[0],
                    dst_ref=recv_hbm.at[u - 1, h],
                    send_sem=send_sems.at[0],
                    recv_sem=recv_sems.at[u - 1, h],
                    device_id=(my,),
                    device_id_type=pl.DeviceIdType.MESH,
                )
                recv.wait_recv()
                local_copy(recv_hbm.at[u - 1, h], send_buf.at[0])
                acc[...] = send_buf[0].astype(jnp.float32)
                local_copy(acc,
                           out_hbm.at[pl.ds(s * M_SH + h * MH, MH), :])

    out, _ = pl.pallas_call(
        body,
        out_shape=[
            jax.ShapeDtypeStruct((N_DEV * M_SH, N_PER), jnp.float32),
            jax.ShapeDtypeStruct((3, 2, MH, N_PER), jnp.bfloat16),
        ],
        in_specs=[
            pl.BlockSpec(memory_space=pltpu.MemorySpace.HBM),
            pl.BlockSpec(memory_space=pltpu.MemorySpace.HBM),
        ],
        out_specs=[
            pl.BlockSpec(memory_space=pltpu.MemorySpace.HBM),
            pl.BlockSpec(memory_space=pltpu.MemorySpace.HBM),
        ],
        scratch_shapes=[
            pltpu.VMEM((MH, K), jnp.bfloat16),
            pltpu.VMEM((2, WT, N_PER), jnp.float32),
            pltpu.VMEM((DT, N_PER), jnp.bfloat16),
            pltpu.VMEM((MH, N_PER), jnp.float32),
            pltpu.VMEM((2, MH, XT), jnp.float32),
            pltpu.VMEM((2, MH, N_PER), jnp.bfloat16),
            pltpu.SemaphoreType.DMA,
            pltpu.SemaphoreType.DMA((2,)),
            pltpu.SemaphoreType.DMA((2,)),
            pltpu.SemaphoreType.DMA((2,)),
            pltpu.SemaphoreType.DMA((3, 2)),
        ],
    )(x, w_mat)
    return out


# device time: 306963 ns/iter; 1.5275x vs baseline; 1.1183x over previous
import jax
import jax.numpy as jnp
from jax import lax
from jax.experimental import pallas as pl
from jax.experimental.pallas import tpu as pltpu

N_DEV = 4
M_SH = 2048
K = 8192
N = 4096
N_PER = N // N_DEV
MH = M_SH // 2
WT = 512
NWT = K // WT
DT = 512
SUB = DT // WT
XT = 256
NXT = K // XT


def kernel(x, w_mat):
    def body(x_hbm, w_hbm, out_hbm, recv_hbm,
             x_bf, w_stage, w_bf, acc, x_stage, send_buf,
             copy_sem, wsems, xsems, send_sems, recv_sems):
        my = lax.axis_index("i")

        def local_copy(src, dst):
            cp = pltpu.make_async_copy(src, dst, copy_sem)
            cp.start()
            cp.wait()

        pending = {0: None, 1: None}

        for h in range(2):
            def xcp(kt, slot):
                return pltpu.make_async_copy(
                    x_hbm.at[pl.ds(h * MH, MH), pl.ds(kt * XT, XT)],
                    x_stage.at[slot], xsems.at[slot])

            xcp(0, 0).start()

            def xconv(kt, carry):
                slot = lax.rem(kt, 2)

                @pl.when(kt + 1 < NXT)
                def _():
                    xcp(kt + 1, 1 - slot).start()

                xcp(kt, slot).wait()
                x_bf[:, pl.ds(kt * XT, XT)] = (
                    x_stage[slot].astype(jnp.bfloat16))
                return carry

            lax.fori_loop(0, NXT, xconv, 0)

            for t in (1, 2, 3, 0):
                j = (my + t) % N_DEV
                col = pl.ds(j * N_PER, N_PER)

                def wcp(kt, slot, col=col):
                    return pltpu.make_async_copy(
                        w_hbm.at[pl.ds(kt * WT, WT), col],
                        w_stage.at[slot], wsems.at[slot])

                def convert(kt, slot, q):
                    wcp(kt, slot).wait()
                    w_bf[pl.ds(q * WT, WT), :] = (
                        w_stage[slot].astype(jnp.bfloat16))

                wcp(0, 0).start()

                for q in range(SUB):
                    wcp(q + 1, (q + 1) % 2).start()
                    convert(q, q % 2, q)
                acc[...] = jnp.dot(
                    x_bf[:, pl.ds(0, DT)], w_bf[...],
                    preferred_element_type=jnp.float32)

                def kstep(kk, carry):
                    for q in range(SUB):
                        kt = SUB * kk + q
                        slot = lax.rem(kt, 2)

                        @pl.when(kt + 1 < NWT)
                        def _():
                            wcp(kt + 1, 1 - slot).start()

                        convert(kt, slot, q)
                    acc[...] = acc[...] + jnp.dot(
                        x_bf[:, pl.ds(kk * DT, DT)], w_bf[...],
                        preferred_element_type=jnp.float32)
                    return carry

                lax.fori_loop(1, K // DT, kstep, 0, unroll=2)

                if t == 0:
                    acc[...] = jnp.maximum(acc[...], 0.0)
                    local_copy(acc,
                               out_hbm.at[pl.ds(my * M_SH + h * MH, MH), :])
                else:
                    s = (t - 1) % 2
                    if pending[s] is not None:
                        pending[s].wait_send()
                    send_buf[s] = jnp.maximum(acc[...], 0.0).astype(
                        jnp.bfloat16)
                    rdma = pltpu.make_async_remote_copy(
                        src_ref=send_buf.at[s],
                        dst_ref=recv_hbm.at[3 - t, h],
                        send_sem=send_sems.at[s],
                        recv_sem=recv_sems.at[3 - t, h],
                        device_id=(j,),
                        device_id_type=pl.DeviceIdType.MESH,
                    )
                    rdma.start()
                    pending[s] = rdma

        for s in (0, 1):
            if pending[s] is not None:
                pending[s].wait_send()
                pending[s] = None

        for h in range(2):
            for u in (3, 2, 1):
                s = (my + u) % N_DEV
                recv = pltpu.make_async_remote_copy(
                    src_ref=send_buf.at[0],
                    dst_ref=recv_hbm.at[u - 1, h],
                    send_sem=send_sems.at[0],
                    recv_sem=recv_sems.at[u - 1, h],
                    device_id=(my,),
                    device_id_type=pl.DeviceIdType.MESH,
                )
                recv.wait_recv()
                local_copy(recv_hbm.at[u - 1, h], send_buf.at[0])
                acc[...] = send_buf[0].astype(jnp.float32)
                local_copy(acc,
                           out_hbm.at[pl.ds(s * M_SH + h * MH, MH), :])

    out, _ = pl.pallas_call(
        body,
        out_shape=[
            jax.ShapeDtypeStruct((N_DEV * M_SH, N_PER), jnp.float32),
            jax.ShapeDtypeStruct((3, 2, MH, N_PER), jnp.bfloat16),
        ],
        in_specs=[
            pl.BlockSpec(memory_space=pltpu.MemorySpace.HBM),
            pl.BlockSpec(memory_space=pltpu.MemorySpace.HBM),
        ],
        out_specs=[
            pl.BlockSpec(memory_space=pltpu.MemorySpace.HBM),
            pl.BlockSpec(memory_space=pltpu.MemorySpace.HBM),
        ],
        scratch_shapes=[
            pltpu.VMEM((MH, K), jnp.bfloat16),
            pltpu.VMEM((2, WT, N_PER), jnp.float32),
            pltpu.VMEM((DT, N_PER), jnp.bfloat16),
            pltpu.VMEM((MH, N_PER), jnp.float32),
            pltpu.VMEM((2, MH, XT), jnp.float32),
            pltpu.VMEM((2, MH, N_PER), jnp.bfloat16),
            pltpu.SemaphoreType.DMA,
            pltpu.SemaphoreType.DMA((2,)),
            pltpu.SemaphoreType.DMA((2,)),
            pltpu.SemaphoreType.DMA((2,)),
            pltpu.SemaphoreType.DMA((3, 2)),
        ],
    )(x, w_mat)
    return out


# device time: 302419 ns/iter; 1.5505x vs baseline; 1.0150x over previous
import jax
import jax.numpy as jnp
from jax import lax
from jax.experimental import pallas as pl
from jax.experimental.pallas import tpu as pltpu

N_DEV = 4
M_SH = 2048
K = 8192
N = 4096
N_PER = N // N_DEV
MH = M_SH // 2
WT = 512
NWT = K // WT
DT = 512
SUB = DT // WT
XT = 256
NXT = K // XT


def kernel(x, w_mat):
    def body(x_hbm, w_hbm, out_hbm, recv_hbm,
             x_bf, w_stage, w_bf, acc, x_stage, send_buf,
             copy_sem, wsems, xsems, send_sems, recv_sems):
        my = lax.axis_index("i")

        def local_copy(src, dst):
            cp = pltpu.make_async_copy(src, dst, copy_sem)
            cp.start()
            cp.wait()

        pending = {0: None, 1: None}

        for h in range(2):
            def xcp(kt, slot):
                return pltpu.make_async_copy(
                    x_hbm.at[pl.ds(h * MH, MH), pl.ds(kt * XT, XT)],
                    x_stage.at[slot], xsems.at[slot])

            xcp(0, 0).start()

            def xconv(kt, carry):
                slot = lax.rem(kt, 2)

                @pl.when(kt + 1 < NXT)
                def _():
                    xcp(kt + 1, 1 - slot).start()

                xcp(kt, slot).wait()
                x_bf[:, pl.ds(kt * XT, XT)] = (
                    x_stage[slot].astype(jnp.bfloat16))
                return carry

            lax.fori_loop(0, NXT, xconv, 0)

            for t in (1, 2, 3, 0):
                j = (my + t) % N_DEV
                col = pl.ds(j * N_PER, N_PER)

                def wcp(kt, slot, col=col):
                    return pltpu.make_async_copy(
                        w_hbm.at[pl.ds(kt * WT, WT), col],
                        w_stage.at[slot], wsems.at[slot])

                def convert(kt, slot, q):
                    wcp(kt, slot).wait()
                    w_bf[pl.ds(q * WT, WT), :] = (
                        w_stage[slot].astype(jnp.bfloat16))

                wcp(0, 0).start()

                for q in range(SUB):
                    wcp(q + 1, (q + 1) % 2).start()
                    convert(q, q % 2, q)
                acc[...] = jnp.dot(
                    x_bf[:, pl.ds(0, DT)], w_bf[...],
                    preferred_element_type=jnp.float32)

                def kstep(kk, carry):
                    for q in range(SUB):
                        kt = SUB * kk + q
                        slot = lax.rem(kt, 2)

                        @pl.when(kt + 1 < NWT)
                        def _():
                            wcp(kt + 1, 1 - slot).start()

                        convert(kt, slot, q)
                    acc[...] = acc[...] + jnp.dot(
                        x_bf[:, pl.ds(kk * DT, DT)], w_bf[...],
                        preferred_element_type=jnp.float32)
                    return carry

                lax.fori_loop(1, K // DT, kstep, 0, unroll=2)

                if t == 0:
                    acc[...] = jnp.maximum(acc[...], 0.0)
                    local_copy(acc,
                               out_hbm.at[pl.ds(my * M_SH + h * MH, MH), :])
                else:
                    s = (t - 1) % 2
                    if pending[s] is not None:
                        pending[s].wait_send()
                    send_buf[s] = jnp.maximum(acc[...], 0.0).astype(
                        jnp.bfloat16)
                    rdma = pltpu.make_async_remote_copy(
                        src_ref=send_buf.at[s],
                        dst_ref=recv_hbm.at[3 - t, h],
                        send_sem=send_sems.at[s],
                        recv_sem=recv_sems.at[3 - t, h],
                        device_id=(j,),
                        device_id_type=pl.DeviceIdType.MESH,
                    )
                    rdma.start()
                    pending[s] = rdma

        for s in (0, 1):
            if pending[s] is not None:
                pending[s].wait_send()
                pending[s] = None

        chunks = [(u, h) for h in range(2) for u in (3, 2, 1)]

        def chunk_in(i):
            u, h = chunks[i]
            return pltpu.make_async_copy(
                recv_hbm.at[u - 1, h], send_buf.at[i % 2], xsems.at[i % 2])

        def wait_chunk(i):
            u, h = chunks[i]
            recv = pltpu.make_async_remote_copy(
                src_ref=send_buf.at[0],
                dst_ref=recv_hbm.at[u - 1, h],
                send_sem=send_sems.at[0],
                recv_sem=recv_sems.at[u - 1, h],
                device_id=(my,),
                device_id_type=pl.DeviceIdType.MESH,
            )
            recv.wait_recv()

        wait_chunk(0)
        chunk_in(0).start()
        for i, (u, h) in enumerate(chunks):
            chunk_in(i).wait()
            if i + 1 < len(chunks):
                wait_chunk(i + 1)
                chunk_in(i + 1).start()
            s = (my + u) % N_DEV
            row0 = s * M_SH + h * MH
            outs = []
            for half in range(2):
                w_stage[half] = (
                    send_buf[i % 2, pl.ds(half * WT, WT), :]
                    .astype(jnp.float32))
                cp = pltpu.make_async_copy(
                    w_stage.at[half],
                    out_hbm.at[pl.ds(row0 + half * WT, WT), :],
                    wsems.at[half])
                cp.start()
                outs.append(cp)
            for cp in outs:
                cp.wait()

    out, _ = pl.pallas_call(
        body,
        out_shape=[
            jax.ShapeDtypeStruct((N_DEV * M_SH, N_PER), jnp.float32),
            jax.ShapeDtypeStruct((3, 2, MH, N_PER), jnp.bfloat16),
        ],
        in_specs=[
            pl.BlockSpec(memory_space=pltpu.MemorySpace.HBM),
            pl.BlockSpec(memory_space=pltpu.MemorySpace.HBM),
        ],
        out_specs=[
            pl.BlockSpec(memory_space=pltpu.MemorySpace.HBM),
            pl.BlockSpec(memory_space=pltpu.MemorySpace.HBM),
        ],
        scratch_shapes=[
            pltpu.VMEM((MH, K), jnp.bfloat16),
            pltpu.VMEM((2, WT, N_PER), jnp.float32),
            pltpu.VMEM((DT, N_PER), jnp.bfloat16),
            pltpu.VMEM((MH, N_PER), jnp.float32),
            pltpu.VMEM((2, MH, XT), jnp.float32),
            pltpu.VMEM((2, MH, N_PER), jnp.bfloat16),
            pltpu.SemaphoreType.DMA,
            pltpu.SemaphoreType.DMA((2,)),
            pltpu.SemaphoreType.DMA((2,)),
            pltpu.SemaphoreType.DMA((2,)),
            pltpu.SemaphoreType.DMA((3, 2)),
        ],
    )(x, w_mat)
    return out
